# baseline (speedup 1.0000x reference)
import functools

import jax
import jax.numpy as jnp
from jax import lax
from jax.experimental import pallas as pl
from jax.experimental.pallas import tpu as pltpu
from jax.experimental.pallas import tpu_sc as plsc

NC = 2
NS = 16
NW = NC * NS
LANES = 16
CHUNK = 64


def _cdiv(a, b):
    return (a + b - 1) // b


def _make_deg_kernel(n_pad, k):
    stripe = n_pad // NS
    mesh = plsc.VectorSubcoreMesh(core_axis_name="c", subcore_axis_name="s")

    @functools.partial(
        pl.kernel,
        mesh=mesh,
        out_type=jax.ShapeDtypeStruct((NC, n_pad), jnp.float32),
        scratch_types=[
            pltpu.VMEM((k, CHUNK), jnp.int32),
            pltpu.VMEM((CHUNK,), jnp.float32),
            pltpu.VMEM_SHARED((n_pad,), jnp.float32),
        ],
    )
    def deg_kernel(dst_hbm, zeros_hbm, out_hbm, dst_v, ones_v, acc_sh):
        c = lax.axis_index("c")
        s = lax.axis_index("s")
        wid = s * NC + c
        pltpu.sync_copy(dst_hbm.at[wid], dst_v)
        for i in range(CHUNK // LANES):
            ones_v[pl.ds(i * LANES, LANES)] = jnp.ones((LANES,), jnp.float32)
        pltpu.sync_copy(zeros_hbm.at[pl.ds(s * stripe, stripe)],
                        acc_sh.at[pl.ds(s * stripe, stripe)])
        plsc.subcore_barrier()

        def body(j, carry):
            pltpu.sync_copy(ones_v, acc_sh.at[dst_v.at[j]], add=True)
            return carry

        lax.fori_loop(0, k, body, 0)
        plsc.subcore_barrier()
        pltpu.sync_copy(acc_sh.at[pl.ds(s * stripe, stripe)],
                        out_hbm.at[c, pl.ds(s * stripe, stripe)])

    return deg_kernel


NBUF = 3
GLAG = 2


def _make_scatter_kernel(n_pad, f, k):
    stripe = n_pad // NS
    groups = k // NBUF
    mesh = plsc.VectorSubcoreMesh(core_axis_name="c", subcore_axis_name="s")

    @functools.partial(
        pl.kernel,
        mesh=mesh,
        compiler_params=pltpu.CompilerParams(use_tc_tiling_on_sc=False),
        out_type=jax.ShapeDtypeStruct((NC, n_pad, f), jnp.float32),
        scratch_types=[
            pltpu.VMEM((k, CHUNK), jnp.int32),
            pltpu.VMEM((k, CHUNK), jnp.int32),
            [pltpu.VMEM((CHUNK, f), jnp.float32) for _ in range(NBUF)],
            pltpu.VMEM_SHARED((n_pad, f), jnp.float32),
            [pltpu.SemaphoreType.DMA for _ in range(NBUF)],
            [pltpu.SemaphoreType.DMA for _ in range(NBUF)],
        ],
    )
    def scatter_kernel(src_hbm, dst_hbm, g_hbm, zeros_hbm, out_hbm,
                       src_v, dst_v, rows, acc_sh, sem_g, sem_s):
        c = lax.axis_index("c")
        s = lax.axis_index("s")
        wid = s * NC + c
        pltpu.sync_copy(src_hbm.at[wid], src_v)
        pltpu.sync_copy(dst_hbm.at[wid], dst_v)
        pltpu.sync_copy(zeros_hbm.at[pl.ds(s * stripe, stripe)],
                        acc_sh.at[pl.ds(s * stripe, stripe)])
        plsc.subcore_barrier()

        def fire_g(j, t):
            pltpu.async_copy(g_hbm.at[src_v.at[j]], rows[t], sem_g[t])

        def wait_g(j, t):
            pltpu.make_async_copy(g_hbm.at[src_v.at[j]], rows[t],
                                  sem_g[t]).wait()

        def fire_s(j, t):
            pass

        def wait_s(j, t):
            pass

        for t in range(GLAG):
            fire_g(t, t)

        def body(g, carry):
            for t in range(NBUF):
                j = g * NBUF + t
                wait_g(j, t)
                fire_s(j, t)
                u = (t + GLAG) % NBUF
                jn = j + GLAG
                jd = jn - NBUF

                @pl.when(jd >= 0)
                def _():
                    wait_s(jd, u)

                @pl.when(jn < k)
                def _():
                    fire_g(jn, u)
            return carry

        lax.fori_loop(0, groups, body, 0)
        for j in range(k - (NBUF - GLAG), k):
            wait_s(j, j % NBUF)
        plsc.subcore_barrier()
        pltpu.sync_copy(acc_sh.at[pl.ds(s * stripe, stripe)],
                        out_hbm.at[c, pl.ds(s * stripe, stripe)])

    return scatter_kernel


BM = 512


def _g_body(x_ref, w_ref, degp_ref, o_ref):
    d = lax.rsqrt(degp_ref[0, :] + degp_ref[1, :] + 1.0)
    h = jnp.dot(x_ref[...], w_ref[...], preferred_element_type=jnp.float32)
    o_ref[...] = h * d[:, None]


def _mid_body(p_ref, g1_ref, degp_ref, b1_ref, w2_ref, o_ref):
    d = lax.rsqrt(degp_ref[0, :] + degp_ref[1, :] + 1.0)
    s_tot = p_ref[0] + p_ref[1] + g1_ref[...]
    out1 = jnp.maximum(s_tot * d[:, None] + b1_ref[...], 0.0)
    o_ref[...] = jnp.dot(out1, w2_ref[...],
                         preferred_element_type=jnp.float32) * d[:, None]


def _fin_body(q_ref, g2_ref, degp_ref, b2_ref, o_ref):
    d = lax.rsqrt(degp_ref[0, :] + degp_ref[1, :] + 1.0)
    s_tot = q_ref[0] + q_ref[1] + g2_ref[...]
    o_ref[...] = jnp.maximum(s_tot * d[:, None] + b2_ref[...], 0.0)


def _row_spec(f):
    return pl.BlockSpec((BM, f), lambda i: (i, 0))


def _part_spec(f):
    return pl.BlockSpec((NC, BM, f), lambda i: (0, i, 0))


_DEG_SPEC = pl.BlockSpec((NC, BM), lambda i: (0, i))


def _full_spec(r, c):
    return pl.BlockSpec((r, c), lambda i: (0, 0))


def kernel(x, edge_index, W1, b1, W2, b2):
    N, F_in = x.shape
    H = W1.shape[1]
    C = W2.shape[1]
    E = edge_index.shape[1]

    n_pad = _cdiv(N, BM) * BM
    k = _cdiv(_cdiv(E, NW * CHUNK), NBUF) * NBUF
    e_pad = NW * k * CHUNK

    pad_node = n_pad - 1
    src = jnp.full((e_pad,), pad_node, jnp.int32).at[:E].set(
        edge_index[0].astype(jnp.int32)).reshape(NW, k, CHUNK)
    dst = jnp.full((e_pad,), pad_node, jnp.int32).at[:E].set(
        edge_index[1].astype(jnp.int32)).reshape(NW, k, CHUNK)
    x_pad = jnp.zeros((n_pad, F_in), jnp.float32).at[:N].set(x)

    zeros1 = jnp.zeros((n_pad,), jnp.float32)
    zerosH = jnp.zeros((n_pad, H), jnp.float32)
    zerosC = jnp.zeros((n_pad, C), jnp.float32)

    degp = _make_deg_kernel(n_pad, k)(dst, zeros1)

    grid = (n_pad // BM,)
    g1 = pl.pallas_call(
        _g_body,
        grid=grid,
        in_specs=[_row_spec(F_in), _full_spec(F_in, H), _DEG_SPEC],
        out_specs=_row_spec(H),
        out_shape=jax.ShapeDtypeStruct((n_pad, H), jnp.float32),
    )(x_pad, W1, degp)

    p1 = _make_scatter_kernel(n_pad, H, k)(src, dst, g1, zerosH)

    g2 = pl.pallas_call(
        _mid_body,
        grid=grid,
        in_specs=[_part_spec(H), _row_spec(H), _DEG_SPEC,
                  _full_spec(1, H), _full_spec(H, C)],
        out_specs=_row_spec(C),
        out_shape=jax.ShapeDtypeStruct((n_pad, C), jnp.float32),
    )(p1, g1, degp, b1.reshape(1, H), W2)

    p2 = _make_scatter_kernel(n_pad, C, k)(src, dst, g2, zerosC)

    out = pl.pallas_call(
        _fin_body,
        grid=grid,
        in_specs=[_part_spec(C), _row_spec(C), _DEG_SPEC, _full_spec(1, C)],
        out_specs=_row_spec(C),
        out_shape=jax.ShapeDtypeStruct((n_pad, C), jnp.float32),
    )(p2, g2, degp, b2.reshape(1, C))

    return out[:N]


# speedup vs baseline: 1.2262x; 1.2262x over previous
import functools

import jax
import jax.numpy as jnp
from jax import lax
from jax.experimental import pallas as pl
from jax.experimental.pallas import tpu as pltpu
from jax.experimental.pallas import tpu_sc as plsc

NC = 2
NS = 16
NW = NC * NS
LANES = 16
CHUNK = 64


def _cdiv(a, b):
    return (a + b - 1) // b


def _make_deg_kernel(n_pad, k):
    stripe = n_pad // NS
    mesh = plsc.VectorSubcoreMesh(core_axis_name="c", subcore_axis_name="s")

    @functools.partial(
        pl.kernel,
        mesh=mesh,
        out_type=jax.ShapeDtypeStruct((NC, n_pad), jnp.float32),
        scratch_types=[
            pltpu.VMEM((k, CHUNK), jnp.int32),
            pltpu.VMEM((CHUNK,), jnp.float32),
            pltpu.VMEM_SHARED((n_pad,), jnp.float32),
        ],
    )
    def deg_kernel(dst_hbm, zeros_hbm, out_hbm, dst_v, ones_v, acc_sh):
        c = lax.axis_index("c")
        s = lax.axis_index("s")
        wid = s * NC + c
        pltpu.sync_copy(dst_hbm.at[wid], dst_v)
        for i in range(CHUNK // LANES):
            ones_v[pl.ds(i * LANES, LANES)] = jnp.ones((LANES,), jnp.float32)
        pltpu.sync_copy(zeros_hbm.at[pl.ds(s * stripe, stripe)],
                        acc_sh.at[pl.ds(s * stripe, stripe)])
        plsc.subcore_barrier()

        def body(j, carry):
            pltpu.sync_copy(ones_v, acc_sh.at[dst_v.at[j]], add=True)
            return carry

        lax.fori_loop(0, k, body, 0)
        plsc.subcore_barrier()
        pltpu.sync_copy(acc_sh.at[pl.ds(s * stripe, stripe)],
                        out_hbm.at[c, pl.ds(s * stripe, stripe)])

    return deg_kernel


NBUF = 3
GLAG = 2


def _make_scatter_kernel(n_pad, f, k, staged):
    stripe = n_pad // NS
    groups = k // NBUF
    mesh = plsc.VectorSubcoreMesh(core_axis_name="c", subcore_axis_name="s")

    scratch = [
        pltpu.VMEM((k, CHUNK), jnp.int32),
        pltpu.VMEM((k, CHUNK), jnp.int32),
        [pltpu.VMEM((CHUNK, f), jnp.float32) for _ in range(NBUF)],
        pltpu.VMEM_SHARED((n_pad, f), jnp.float32),
        [pltpu.SemaphoreType.DMA for _ in range(NBUF)],
        [pltpu.SemaphoreType.DMA for _ in range(NBUF)],
    ]
    if staged:
        scratch.append(pltpu.VMEM_SHARED((n_pad, f), jnp.float32))

    @functools.partial(
        pl.kernel,
        mesh=mesh,
        compiler_params=pltpu.CompilerParams(use_tc_tiling_on_sc=False),
        out_type=jax.ShapeDtypeStruct((NC, n_pad, f), jnp.float32),
        scratch_types=scratch,
    )
    def scatter_kernel(src_hbm, dst_hbm, g_hbm, zeros_hbm, out_hbm,
                       src_v, dst_v, rows, acc_sh, sem_g, sem_s, *maybe_gsh):
        c = lax.axis_index("c")
        s = lax.axis_index("s")
        wid = s * NC + c
        pltpu.sync_copy(src_hbm.at[wid], src_v)
        pltpu.sync_copy(dst_hbm.at[wid], dst_v)
        pltpu.sync_copy(zeros_hbm.at[pl.ds(s * stripe, stripe)],
                        acc_sh.at[pl.ds(s * stripe, stripe)])
        if staged:
            g_tbl = maybe_gsh[0]
            pltpu.sync_copy(g_hbm.at[pl.ds(s * stripe, stripe)],
                            g_tbl.at[pl.ds(s * stripe, stripe)])
        else:
            g_tbl = g_hbm
        plsc.subcore_barrier()

        def fire_g(j, t):
            pltpu.async_copy(g_tbl.at[src_v.at[j]], rows[t], sem_g[t])

        def wait_g(j, t):
            pltpu.make_async_copy(g_tbl.at[src_v.at[j]], rows[t],
                                  sem_g[t]).wait()

        def fire_s(j, t):
            pltpu.async_copy(rows[t], acc_sh.at[dst_v.at[j]],
                             sem_s[t], add=True)

        def wait_s(j, t):
            pltpu.make_async_copy(rows[t], acc_sh.at[dst_v.at[j]],
                                  sem_s[t]).wait()

        for t in range(GLAG):
            fire_g(t, t)

        def body(g, carry):
            for t in range(NBUF):
                j = g * NBUF + t
                wait_g(j, t)
                fire_s(j, t)
                u = (t + GLAG) % NBUF
                jn = j + GLAG
                jd = jn - NBUF

                @pl.when(jd >= 0)
                def _():
                    wait_s(jd, u)

                @pl.when(jn < k)
                def _():
                    fire_g(jn, u)
            return carry

        lax.fori_loop(0, groups, body, 0)
        for j in range(k - (NBUF - GLAG), k):
            wait_s(j, j % NBUF)
        plsc.subcore_barrier()
        pltpu.sync_copy(acc_sh.at[pl.ds(s * stripe, stripe)],
                        out_hbm.at[c, pl.ds(s * stripe, stripe)])

    return scatter_kernel


BM = 512


def _g_body(x_ref, w_ref, degp_ref, o_ref):
    d = lax.rsqrt(degp_ref[0, :] + degp_ref[1, :] + 1.0)
    h = jnp.dot(x_ref[...], w_ref[...], preferred_element_type=jnp.float32)
    o_ref[...] = h * d[:, None]


def _mid_body(p_ref, g1_ref, degp_ref, b1_ref, w2_ref, o_ref):
    d = lax.rsqrt(degp_ref[0, :] + degp_ref[1, :] + 1.0)
    s_tot = p_ref[0] + p_ref[1] + g1_ref[...]
    out1 = jnp.maximum(s_tot * d[:, None] + b1_ref[...], 0.0)
    o_ref[...] = jnp.dot(out1, w2_ref[...],
                         preferred_element_type=jnp.float32) * d[:, None]


def _fin_body(q_ref, g2_ref, degp_ref, b2_ref, o_ref):
    d = lax.rsqrt(degp_ref[0, :] + degp_ref[1, :] + 1.0)
    s_tot = q_ref[0] + q_ref[1] + g2_ref[...]
    o_ref[...] = jnp.maximum(s_tot * d[:, None] + b2_ref[...], 0.0)


def _row_spec(f):
    return pl.BlockSpec((BM, f), lambda i: (i, 0))


def _part_spec(f):
    return pl.BlockSpec((NC, BM, f), lambda i: (0, i, 0))


_DEG_SPEC = pl.BlockSpec((NC, BM), lambda i: (0, i))


def _full_spec(r, c):
    return pl.BlockSpec((r, c), lambda i: (0, 0))


def kernel(x, edge_index, W1, b1, W2, b2):
    N, F_in = x.shape
    H = W1.shape[1]
    C = W2.shape[1]
    E = edge_index.shape[1]

    n_pad = _cdiv(N, BM) * BM
    k = _cdiv(_cdiv(E, NW * CHUNK), NBUF) * NBUF
    e_pad = NW * k * CHUNK

    pad_node = n_pad - 1
    src = jnp.full((e_pad,), pad_node, jnp.int32).at[:E].set(
        edge_index[0].astype(jnp.int32)).reshape(NW, k, CHUNK)
    dst = jnp.full((e_pad,), pad_node, jnp.int32).at[:E].set(
        edge_index[1].astype(jnp.int32)).reshape(NW, k, CHUNK)
    x_pad = jnp.zeros((n_pad, F_in), jnp.float32).at[:N].set(x)

    zeros1 = jnp.zeros((n_pad,), jnp.float32)
    zerosH = jnp.zeros((n_pad, H), jnp.float32)
    zerosC = jnp.zeros((n_pad, C), jnp.float32)

    degp = _make_deg_kernel(n_pad, k)(dst, zeros1)

    grid = (n_pad // BM,)
    g1 = pl.pallas_call(
        _g_body,
        grid=grid,
        in_specs=[_row_spec(F_in), _full_spec(F_in, H), _DEG_SPEC],
        out_specs=_row_spec(H),
        out_shape=jax.ShapeDtypeStruct((n_pad, H), jnp.float32),
    )(x_pad, W1, degp)

    p1 = _make_scatter_kernel(n_pad, H, k, staged=False)(src, dst, g1, zerosH)

    g2 = pl.pallas_call(
        _mid_body,
        grid=grid,
        in_specs=[_part_spec(H), _row_spec(H), _DEG_SPEC,
                  _full_spec(1, H), _full_spec(H, C)],
        out_specs=_row_spec(C),
        out_shape=jax.ShapeDtypeStruct((n_pad, C), jnp.float32),
    )(p1, g1, degp, b1.reshape(1, H), W2)

    p2 = _make_scatter_kernel(n_pad, C, k, staged=True)(src, dst, g2, zerosC)

    out = pl.pallas_call(
        _fin_body,
        grid=grid,
        in_specs=[_part_spec(C), _row_spec(C), _DEG_SPEC, _full_spec(1, C)],
        out_specs=_row_spec(C),
        out_shape=jax.ShapeDtypeStruct((n_pad, C), jnp.float32),
    )(p2, g2, degp, b2.reshape(1, C))

    return out[:N]


# speedup vs baseline: 1.5908x; 1.2973x over previous
import functools

import jax
import jax.numpy as jnp
from jax import lax
from jax.experimental import pallas as pl
from jax.experimental.pallas import tpu as pltpu
from jax.experimental.pallas import tpu_sc as plsc

NC = 2
NS = 16
NW = NC * NS
LANES = 16
CHUNK = 128
FH = 64
NBUF = 3
GLAG = 2


def _cdiv(a, b):
    return (a + b - 1) // b


def _make_deg_kernel(n_pad, k):
    stripe = n_pad // NS
    mesh = plsc.VectorSubcoreMesh(core_axis_name="c", subcore_axis_name="s")

    @functools.partial(
        pl.kernel,
        mesh=mesh,
        out_type=jax.ShapeDtypeStruct((NC, n_pad), jnp.float32),
        scratch_types=[
            pltpu.VMEM((k, CHUNK), jnp.int32),
            pltpu.VMEM((CHUNK,), jnp.float32),
            pltpu.VMEM_SHARED((n_pad,), jnp.float32),
        ],
    )
    def deg_kernel(dst_hbm, zeros_hbm, out_hbm, dst_v, ones_v, acc_sh):
        c = lax.axis_index("c")
        s = lax.axis_index("s")
        wid = s * NC + c
        pltpu.sync_copy(dst_hbm.at[wid], dst_v)
        for i in range(CHUNK // LANES):
            ones_v[pl.ds(i * LANES, LANES)] = jnp.ones((LANES,), jnp.float32)
        pltpu.sync_copy(zeros_hbm.at[pl.ds(s * stripe, stripe)],
                        acc_sh.at[pl.ds(s * stripe, stripe)])
        plsc.subcore_barrier()

        def body(j, carry):
            pltpu.sync_copy(ones_v, acc_sh.at[dst_v.at[j]], add=True)
            return carry

        lax.fori_loop(0, k, body, 0)
        plsc.subcore_barrier()
        pltpu.sync_copy(acc_sh.at[pl.ds(s * stripe, stripe)],
                        out_hbm.at[c, pl.ds(s * stripe, stripe)])

    return deg_kernel


def _make_scatter_kernel(n_pad, k, npass):
    stripe = n_pad // NS
    groups = k // NBUF
    mesh = plsc.VectorSubcoreMesh(core_axis_name="c", subcore_axis_name="s")

    @functools.partial(
        pl.kernel,
        mesh=mesh,
        compiler_params=pltpu.CompilerParams(use_tc_tiling_on_sc=False),
        out_type=jax.ShapeDtypeStruct((npass, NC, n_pad, FH), jnp.float32),
        scratch_types=[
            pltpu.VMEM((k, CHUNK), jnp.int32),
            pltpu.VMEM((k, CHUNK), jnp.int32),
            [pltpu.VMEM((CHUNK, FH), jnp.float32) for _ in range(NBUF)],
            pltpu.VMEM_SHARED((n_pad, FH), jnp.float32),
            pltpu.VMEM_SHARED((n_pad, FH), jnp.float32),
            [pltpu.SemaphoreType.DMA for _ in range(NBUF)],
            [pltpu.SemaphoreType.DMA for _ in range(NBUF)],
        ],
    )
    def scatter_kernel(src_hbm, dst_hbm, g_hbm, zeros_hbm, out_hbm,
                       src_v, dst_v, rows, g_sh, acc_sh, sem_g, sem_s):
        c = lax.axis_index("c")
        s = lax.axis_index("s")
        wid = s * NC + c
        pltpu.sync_copy(src_hbm.at[wid], src_v)
        pltpu.sync_copy(dst_hbm.at[wid], dst_v)

        def fire_g(j, t):
            pltpu.async_copy(g_sh.at[src_v.at[j]], rows[t], sem_g[t])

        def wait_g(j, t):
            pltpu.make_async_copy(g_sh.at[src_v.at[j]], rows[t],
                                  sem_g[t]).wait()

        def fire_s(j, t):
            pltpu.async_copy(rows[t], acc_sh.at[dst_v.at[j]],
                             sem_s[t], add=True)

        def wait_s(j, t):
            pltpu.make_async_copy(rows[t], acc_sh.at[dst_v.at[j]],
                                  sem_s[t]).wait()

        for p in range(npass):
            pltpu.sync_copy(g_hbm.at[p, pl.ds(s * stripe, stripe)],
                            g_sh.at[pl.ds(s * stripe, stripe)])
            pltpu.sync_copy(zeros_hbm.at[pl.ds(s * stripe, stripe)],
                            acc_sh.at[pl.ds(s * stripe, stripe)])
            plsc.subcore_barrier()

            for t in range(GLAG):
                fire_g(t, t)

            def body(g, carry):
                for t in range(NBUF):
                    j = g * NBUF + t
                    wait_g(j, t)
                    fire_s(j, t)
                    u = (t + GLAG) % NBUF
                    jn = j + GLAG
                    jd = jn - NBUF

                    @pl.when(jd >= 0)
                    def _():
                        wait_s(jd, u)

                    @pl.when(jn < k)
                    def _():
                        fire_g(jn, u)
                return carry

            lax.fori_loop(0, groups, body, 0)
            for j in range(k - (NBUF - GLAG), k):
                wait_s(j, j % NBUF)
            plsc.subcore_barrier()
            pltpu.sync_copy(acc_sh.at[pl.ds(s * stripe, stripe)],
                            out_hbm.at[p, c, pl.ds(s * stripe, stripe)])

    return scatter_kernel


BM = 512


def _g1_body(x_ref, w_ref, degp_ref, o_ref):
    d = lax.rsqrt(degp_ref[0, :] + degp_ref[1, :] + 1.0)
    h = jnp.dot(x_ref[...], w_ref[...], preferred_element_type=jnp.float32)
    g = h * d[:, None]
    o_ref[0, :, :] = g[:, :FH]
    o_ref[1, :, :] = g[:, FH:]


def _mid_body(p_ref, g1_ref, degp_ref, b1_ref, w2_ref, o_ref):
    d = lax.rsqrt(degp_ref[0, :] + degp_ref[1, :] + 1.0)
    s_lo = p_ref[0, 0] + p_ref[0, 1] + g1_ref[0]
    s_hi = p_ref[1, 0] + p_ref[1, 1] + g1_ref[1]
    s_tot = jnp.concatenate([s_lo, s_hi], axis=1)
    out1 = jnp.maximum(s_tot * d[:, None] + b1_ref[...], 0.0)
    o_ref[...] = jnp.dot(out1, w2_ref[...],
                         preferred_element_type=jnp.float32) * d[:, None]


def _fin_body(q_ref, g2_ref, degp_ref, b2_ref, o_ref):
    d = lax.rsqrt(degp_ref[0, :] + degp_ref[1, :] + 1.0)
    s_tot = q_ref[0, 0] + q_ref[0, 1] + g2_ref[...]
    o_ref[...] = jnp.maximum(s_tot * d[:, None] + b2_ref[...], 0.0)


def _row_spec(f):
    return pl.BlockSpec((BM, f), lambda i: (i, 0))


def _half_spec(npass):
    return pl.BlockSpec((npass, BM, FH), lambda i: (0, i, 0))


def _part_spec(npass):
    return pl.BlockSpec((npass, NC, BM, FH), lambda i: (0, 0, i, 0))


_DEG_SPEC = pl.BlockSpec((NC, BM), lambda i: (0, i))


def _full_spec(r, c):
    return pl.BlockSpec((r, c), lambda i: (0, 0))


def kernel(x, edge_index, W1, b1, W2, b2):
    N, F_in = x.shape
    H = W1.shape[1]
    C = W2.shape[1]
    E = edge_index.shape[1]

    n_pad = _cdiv(N, BM) * BM
    k = _cdiv(_cdiv(E, NW * CHUNK), NBUF) * NBUF
    e_pad = NW * k * CHUNK

    pad_node = n_pad - 1
    src = jnp.full((e_pad,), pad_node, jnp.int32).at[:E].set(
        edge_index[0].astype(jnp.int32)).reshape(NW, k, CHUNK)
    dst = jnp.full((e_pad,), pad_node, jnp.int32).at[:E].set(
        edge_index[1].astype(jnp.int32)).reshape(NW, k, CHUNK)
    x_pad = jnp.zeros((n_pad, F_in), jnp.float32).at[:N].set(x)

    zeros1 = jnp.zeros((n_pad,), jnp.float32)
    zerosF = jnp.zeros((n_pad, FH), jnp.float32)

    degp = _make_deg_kernel(n_pad, k)(dst, zeros1)

    grid = (n_pad // BM,)
    g1 = pl.pallas_call(
        _g1_body,
        grid=grid,
        in_specs=[_row_spec(F_in), _full_spec(F_in, H), _DEG_SPEC],
        out_specs=_half_spec(2),
        out_shape=jax.ShapeDtypeStruct((2, n_pad, FH), jnp.float32),
    )(x_pad, W1, degp)

    p1 = _make_scatter_kernel(n_pad, k, 2)(src, dst, g1, zerosF)

    g2 = pl.pallas_call(
        _mid_body,
        grid=grid,
        in_specs=[_part_spec(2), _half_spec(2), _DEG_SPEC,
                  _full_spec(1, H), _full_spec(H, C)],
        out_specs=_row_spec(C),
        out_shape=jax.ShapeDtypeStruct((n_pad, C), jnp.float32),
    )(p1, g1, degp, b1.reshape(1, H), W2)

    p2 = _make_scatter_kernel(n_pad, k, 1)(src, dst,
                                           g2.reshape(1, n_pad, FH), zerosF)

    out = pl.pallas_call(
        _fin_body,
        grid=grid,
        in_specs=[_part_spec(1), _row_spec(C), _DEG_SPEC, _full_spec(1, C)],
        out_specs=_row_spec(C),
        out_shape=jax.ShapeDtypeStruct((n_pad, C), jnp.float32),
    )(p2, g2, degp, b2.reshape(1, C))

    return out[:N]
